# Initial kernel scaffold; baseline (speedup 1.0000x reference)
import jax
import jax.numpy as jnp
from jax.experimental import pallas as pl


def kernel(xyz1, xyz2, points1, points2, conv1_w, conv1_b, bn1_g, bn1_b, conv2_w, conv2_b, bn2_g, bn2_b):
    raise NotImplementedError("write your pallas kernel here")


# speedup vs baseline: 26.7623x; 26.7623x over previous
import functools

import jax
import jax.numpy as jnp
from jax.experimental import pallas as pl
from jax.experimental.pallas import tpu as pltpu

_LRELU_SLOPE = 0.2
_BN_EPS = 1e-5
_W_EPS = 1e-8


def _knn_conv1_body(x1_ref, x2_ref, p1_ref, p2_ref, w1a_ref, w1b_ref, b1_ref,
                    h1_ref, st1_ref, *, n_tiles):
    b = pl.program_id(0)
    t = pl.program_id(1)
    x1 = x1_ref[0]
    x2 = x2_ref[0]
    s = x2.shape[1]
    tn = x1.shape[1]

    x1sq = jnp.sum(x1 * x1, axis=0)
    x2sq = jnp.sum(x2 * x2, axis=0)
    cross = jax.lax.dot_general(x2, x1, (((0,), (0,)), ((), ())),
                                preferred_element_type=jnp.float32)
    dist = x2sq[:, None] + x1sq[None, :] - 2.0 * cross

    iota_s = jax.lax.broadcasted_iota(jnp.int32, (s, tn), 0)
    big = jnp.float32(jnp.inf)
    d = dist
    idxs = []
    recips = []
    for _ in range(3):
        dmin = jnp.min(d, axis=0)
        hit = d == dmin[None, :]
        idx = jnp.min(jnp.where(hit, iota_s, s), axis=0)
        idxs.append(idx)
        recips.append(1.0 / (dmin + _W_EPS))
        d = jnp.where(iota_s == idx[None, :], big, d)
    norm = recips[0] + recips[1] + recips[2]
    w0 = recips[0] / norm
    w1 = recips[1] / norm
    w2 = recips[2] / norm
    zero = jnp.zeros((s, tn), jnp.float32)
    wmat = jnp.where(iota_s == idxs[0][None, :], w0[None, :],
           jnp.where(iota_s == idxs[1][None, :], w1[None, :],
           jnp.where(iota_s == idxs[2][None, :], w2[None, :], zero)))

    interp = jax.lax.dot_general(p2_ref[0], wmat, (((1,), (0,)), ((), ())),
                                 preferred_element_type=jnp.float32)
    h1 = (jax.lax.dot_general(w1a_ref[...], p1_ref[0], (((1,), (0,)), ((), ())),
                              preferred_element_type=jnp.float32)
          + jax.lax.dot_general(w1b_ref[...], interp, (((1,), (0,)), ((), ())),
                                preferred_element_type=jnp.float32)
          + b1_ref[...])
    h1_ref[0] = h1

    c = h1.shape[0]
    psum = jnp.sum(h1.reshape(c, tn // 128, 128), axis=1)
    psq = jnp.sum((h1 * h1).reshape(c, tn // 128, 128), axis=1)

    @pl.when(jnp.logical_and(b == 0, t == 0))
    def _():
        st1_ref[0] = psum
        st1_ref[1] = psq

    @pl.when(jnp.logical_not(jnp.logical_and(b == 0, t == 0)))
    def _():
        st1_ref[0] += psum
        st1_ref[1] += psq


def _bn1_conv2_body(h1_ref, st1_ref, g1_ref, bb1_ref, w2_ref, b2_ref,
                    h2_ref, st2_ref, *, count):
    b = pl.program_id(0)
    t = pl.program_id(1)
    inv = jnp.float32(1.0 / count)
    mean = jnp.sum(st1_ref[0], axis=1, keepdims=True) * inv
    ex2 = jnp.sum(st1_ref[1], axis=1, keepdims=True) * inv
    var = ex2 - mean * mean
    scale = g1_ref[...] / jnp.sqrt(var + _BN_EPS)
    shift = bb1_ref[...] - scale * mean
    a = scale * h1_ref[0] + shift
    a = jnp.where(a >= 0, a, _LRELU_SLOPE * a)
    h2 = jax.lax.dot_general(w2_ref[...], a, (((1,), (0,)), ((), ())),
                             preferred_element_type=jnp.float32) + b2_ref[...]
    h2_ref[0] = h2

    c, tn = h2.shape
    psum = jnp.sum(h2.reshape(c, tn // 128, 128), axis=1)
    psq = jnp.sum((h2 * h2).reshape(c, tn // 128, 128), axis=1)

    @pl.when(jnp.logical_and(b == 0, t == 0))
    def _():
        st2_ref[0] = psum
        st2_ref[1] = psq

    @pl.when(jnp.logical_not(jnp.logical_and(b == 0, t == 0)))
    def _():
        st2_ref[0] += psum
        st2_ref[1] += psq


def _bn2_body(h2_ref, st2_ref, g2_ref, bb2_ref, out_ref, *, count):
    inv = jnp.float32(1.0 / count)
    mean = jnp.sum(st2_ref[0], axis=1, keepdims=True) * inv
    ex2 = jnp.sum(st2_ref[1], axis=1, keepdims=True) * inv
    var = ex2 - mean * mean
    scale = g2_ref[...] / jnp.sqrt(var + _BN_EPS)
    shift = bb2_ref[...] - scale * mean
    a = scale * h2_ref[0] + shift
    out_ref[0] = jnp.where(a >= 0, a, _LRELU_SLOPE * a)


def kernel(xyz1, xyz2, points1, points2, conv1_w, conv1_b, bn1_g, bn1_b,
           conv2_w, conv2_b, bn2_g, bn2_b):
    b, _, n = xyz1.shape
    s = xyz2.shape[2]
    d = points1.shape[1]
    c1 = conv1_w.shape[0]
    c2 = conv2_w.shape[0]
    tn = min(512, n)
    n_tiles = n // tn
    count = b * n

    w1a = conv1_w[:, :d]
    w1b = conv1_w[:, d:]
    b1 = conv1_b.reshape(c1, 1)
    g1 = bn1_g.reshape(c1, 1)
    bb1 = bn1_b.reshape(c1, 1)
    b2 = conv2_b.reshape(c2, 1)
    g2 = bn2_g.reshape(c2, 1)
    bb2 = bn2_b.reshape(c2, 1)

    grid = (b, n_tiles)
    h1, st1 = pl.pallas_call(
        functools.partial(_knn_conv1_body, n_tiles=n_tiles),
        grid=grid,
        in_specs=[
            pl.BlockSpec((1, 3, tn), lambda i, j: (i, 0, j)),
            pl.BlockSpec((1, 3, s), lambda i, j: (i, 0, 0)),
            pl.BlockSpec((1, d, tn), lambda i, j: (i, 0, j)),
            pl.BlockSpec((1, d, s), lambda i, j: (i, 0, 0)),
            pl.BlockSpec((c1, d), lambda i, j: (0, 0)),
            pl.BlockSpec((c1, d), lambda i, j: (0, 0)),
            pl.BlockSpec((c1, 1), lambda i, j: (0, 0)),
        ],
        out_specs=[
            pl.BlockSpec((1, c1, tn), lambda i, j: (i, 0, j)),
            pl.BlockSpec((2, c1, 128), lambda i, j: (0, 0, 0)),
        ],
        out_shape=[
            jax.ShapeDtypeStruct((b, c1, n), jnp.float32),
            jax.ShapeDtypeStruct((2, c1, 128), jnp.float32),
        ],
    )(xyz1, xyz2, points1, points2, w1a, w1b, b1)

    h2, st2 = pl.pallas_call(
        functools.partial(_bn1_conv2_body, count=count),
        grid=grid,
        in_specs=[
            pl.BlockSpec((1, c1, tn), lambda i, j: (i, 0, j)),
            pl.BlockSpec((2, c1, 128), lambda i, j: (0, 0, 0)),
            pl.BlockSpec((c1, 1), lambda i, j: (0, 0)),
            pl.BlockSpec((c1, 1), lambda i, j: (0, 0)),
            pl.BlockSpec((c2, c1), lambda i, j: (0, 0)),
            pl.BlockSpec((c2, 1), lambda i, j: (0, 0)),
        ],
        out_specs=[
            pl.BlockSpec((1, c2, tn), lambda i, j: (i, 0, j)),
            pl.BlockSpec((2, c2, 128), lambda i, j: (0, 0, 0)),
        ],
        out_shape=[
            jax.ShapeDtypeStruct((b, c2, n), jnp.float32),
            jax.ShapeDtypeStruct((2, c2, 128), jnp.float32),
        ],
    )(h1, st1, g1, bb1, conv2_w, b2)

    out = pl.pallas_call(
        functools.partial(_bn2_body, count=count),
        grid=grid,
        in_specs=[
            pl.BlockSpec((1, c2, tn), lambda i, j: (i, 0, j)),
            pl.BlockSpec((2, c2, 128), lambda i, j: (0, 0, 0)),
            pl.BlockSpec((c2, 1), lambda i, j: (0, 0)),
            pl.BlockSpec((c2, 1), lambda i, j: (0, 0)),
        ],
        out_specs=pl.BlockSpec((1, c2, tn), lambda i, j: (i, 0, j)),
        out_shape=jax.ShapeDtypeStruct((b, c2, n), jnp.float32),
    )(h2, st2, g2, bb2)

    return out


# speedup vs baseline: 30.3376x; 1.1336x over previous
import functools

import jax
import jax.numpy as jnp
from jax.experimental import pallas as pl
from jax.experimental.pallas import tpu as pltpu

_LRELU_SLOPE = 0.2
_BN_EPS = 1e-5
_W_EPS = 1e-8


def _knn_conv1_body(x1_ref, x2_ref, p1_ref, p2_ref, w1a_ref, w1b_ref, b1_ref,
                    h1_ref, st1_ref, *, n_tiles):
    b = pl.program_id(0)
    t = pl.program_id(1)
    x1 = x1_ref[0]
    x2 = x2_ref[0]
    s = x2.shape[1]
    tn = x1.shape[1]

    x1sq = jnp.sum(x1 * x1, axis=0, keepdims=True)
    x2sq = jnp.sum(x2 * x2, axis=0, keepdims=True)
    ones_n = jnp.ones((1, tn), jnp.float32)
    ones_s = jnp.ones((1, s), jnp.float32)
    x1aug = jnp.concatenate([-2.0 * x1, ones_n, x1sq], axis=0)
    x2aug = jnp.concatenate([x2, x2sq, ones_s], axis=0)
    dist = jax.lax.dot_general(x2aug, x1aug, (((0,), (0,)), ((), ())),
                               preferred_element_type=jnp.float32)

    big = jnp.float32(jnp.inf)
    m0 = jnp.min(dist, axis=0)
    eq0 = dist == m0[None, :]
    c0 = jnp.sum(jnp.where(eq0, 1.0, 0.0), axis=0)
    d1m = jnp.where(eq0, big, dist)
    m1 = jnp.min(d1m, axis=0)
    eq1 = d1m == m1[None, :]
    c1 = jnp.sum(jnp.where(eq1, 1.0, 0.0), axis=0)
    d2m = jnp.where(eq1, big, d1m)
    m2 = jnp.min(d2m, axis=0)
    eq2 = d2m == m2[None, :]

    c0ge2 = c0 > 1.5
    c0ge3 = c0 > 2.5
    c0eq2 = jnp.logical_and(c0ge2, jnp.logical_not(c0ge3))
    c1ge2 = c1 > 1.5
    d0v = m0
    d1v = jnp.where(c0ge2, m0, m1)
    d2v = jnp.where(c0ge3, m0, jnp.where(jnp.logical_or(c0eq2, c1ge2), m1, m2))
    norm = 1.0 / (d0v + _W_EPS) + 1.0 / (d1v + _W_EPS) + 1.0 / (d2v + _W_EPS)
    wv0 = 1.0 / ((m0 + _W_EPS) * norm)
    wv1 = 1.0 / ((m1 + _W_EPS) * norm)
    wv2 = 1.0 / ((m2 + _W_EPS) * norm)
    use1 = jnp.logical_not(c0ge3)
    use2 = jnp.logical_and(c0 < 1.5, c1 < 1.5)
    wv1 = jnp.where(use1, wv1, 0.0)
    wv2 = jnp.where(use2, wv2, 0.0)
    zero = jnp.zeros((s, tn), jnp.float32)
    wmat = jnp.where(eq0, wv0[None, :],
           jnp.where(eq1, wv1[None, :],
           jnp.where(eq2, wv2[None, :], zero)))

    interp = jax.lax.dot_general(p2_ref[0], wmat, (((1,), (0,)), ((), ())),
                                 preferred_element_type=jnp.float32)
    h1 = (jax.lax.dot_general(w1a_ref[...], p1_ref[0], (((1,), (0,)), ((), ())),
                              preferred_element_type=jnp.float32)
          + jax.lax.dot_general(w1b_ref[...], interp, (((1,), (0,)), ((), ())),
                                preferred_element_type=jnp.float32)
          + b1_ref[...])
    h1_ref[0] = h1

    c = h1.shape[0]
    psum = jnp.sum(h1.reshape(c, tn // 128, 128), axis=1)
    psq = jnp.sum((h1 * h1).reshape(c, tn // 128, 128), axis=1)

    @pl.when(jnp.logical_and(b == 0, t == 0))
    def _():
        st1_ref[0] = psum
        st1_ref[1] = psq

    @pl.when(jnp.logical_not(jnp.logical_and(b == 0, t == 0)))
    def _():
        st1_ref[0] += psum
        st1_ref[1] += psq


def _bn1_conv2_body(h1_ref, st1_ref, g1_ref, bb1_ref, w2_ref, b2_ref,
                    h2_ref, st2_ref, *, count):
    b = pl.program_id(0)
    t = pl.program_id(1)
    inv = jnp.float32(1.0 / count)
    mean = jnp.sum(st1_ref[0], axis=1, keepdims=True) * inv
    ex2 = jnp.sum(st1_ref[1], axis=1, keepdims=True) * inv
    var = ex2 - mean * mean
    scale = g1_ref[...] / jnp.sqrt(var + _BN_EPS)
    shift = bb1_ref[...] - scale * mean
    a = scale * h1_ref[0] + shift
    a = jnp.where(a >= 0, a, _LRELU_SLOPE * a)
    h2 = jax.lax.dot_general(w2_ref[...], a, (((1,), (0,)), ((), ())),
                             preferred_element_type=jnp.float32) + b2_ref[...]
    h2_ref[0] = h2

    c, tn = h2.shape
    psum = jnp.sum(h2.reshape(c, tn // 128, 128), axis=1)
    psq = jnp.sum((h2 * h2).reshape(c, tn // 128, 128), axis=1)

    @pl.when(jnp.logical_and(b == 0, t == 0))
    def _():
        st2_ref[0] = psum
        st2_ref[1] = psq

    @pl.when(jnp.logical_not(jnp.logical_and(b == 0, t == 0)))
    def _():
        st2_ref[0] += psum
        st2_ref[1] += psq


def _bn2_body(h2_ref, st2_ref, g2_ref, bb2_ref, out_ref, *, count):
    inv = jnp.float32(1.0 / count)
    mean = jnp.sum(st2_ref[0], axis=1, keepdims=True) * inv
    ex2 = jnp.sum(st2_ref[1], axis=1, keepdims=True) * inv
    var = ex2 - mean * mean
    scale = g2_ref[...] / jnp.sqrt(var + _BN_EPS)
    shift = bb2_ref[...] - scale * mean
    a = scale * h2_ref[0] + shift
    out_ref[0] = jnp.where(a >= 0, a, _LRELU_SLOPE * a)


def kernel(xyz1, xyz2, points1, points2, conv1_w, conv1_b, bn1_g, bn1_b,
           conv2_w, conv2_b, bn2_g, bn2_b):
    b, _, n = xyz1.shape
    s = xyz2.shape[2]
    d = points1.shape[1]
    c1 = conv1_w.shape[0]
    c2 = conv2_w.shape[0]
    tn = min(512, n)
    n_tiles = n // tn
    count = b * n

    w1a = conv1_w[:, :d]
    w1b = conv1_w[:, d:]
    b1 = conv1_b.reshape(c1, 1)
    g1 = bn1_g.reshape(c1, 1)
    bb1 = bn1_b.reshape(c1, 1)
    b2 = conv2_b.reshape(c2, 1)
    g2 = bn2_g.reshape(c2, 1)
    bb2 = bn2_b.reshape(c2, 1)

    grid = (b, n_tiles)
    h1, st1 = pl.pallas_call(
        functools.partial(_knn_conv1_body, n_tiles=n_tiles),
        grid=grid,
        in_specs=[
            pl.BlockSpec((1, 3, tn), lambda i, j: (i, 0, j)),
            pl.BlockSpec((1, 3, s), lambda i, j: (i, 0, 0)),
            pl.BlockSpec((1, d, tn), lambda i, j: (i, 0, j)),
            pl.BlockSpec((1, d, s), lambda i, j: (i, 0, 0)),
            pl.BlockSpec((c1, d), lambda i, j: (0, 0)),
            pl.BlockSpec((c1, d), lambda i, j: (0, 0)),
            pl.BlockSpec((c1, 1), lambda i, j: (0, 0)),
        ],
        out_specs=[
            pl.BlockSpec((1, c1, tn), lambda i, j: (i, 0, j)),
            pl.BlockSpec((2, c1, 128), lambda i, j: (0, 0, 0)),
        ],
        out_shape=[
            jax.ShapeDtypeStruct((b, c1, n), jnp.float32),
            jax.ShapeDtypeStruct((2, c1, 128), jnp.float32),
        ],
    )(xyz1, xyz2, points1, points2, w1a, w1b, b1)

    h2, st2 = pl.pallas_call(
        functools.partial(_bn1_conv2_body, count=count),
        grid=grid,
        in_specs=[
            pl.BlockSpec((1, c1, tn), lambda i, j: (i, 0, j)),
            pl.BlockSpec((2, c1, 128), lambda i, j: (0, 0, 0)),
            pl.BlockSpec((c1, 1), lambda i, j: (0, 0)),
            pl.BlockSpec((c1, 1), lambda i, j: (0, 0)),
            pl.BlockSpec((c2, c1), lambda i, j: (0, 0)),
            pl.BlockSpec((c2, 1), lambda i, j: (0, 0)),
        ],
        out_specs=[
            pl.BlockSpec((1, c2, tn), lambda i, j: (i, 0, j)),
            pl.BlockSpec((2, c2, 128), lambda i, j: (0, 0, 0)),
        ],
        out_shape=[
            jax.ShapeDtypeStruct((b, c2, n), jnp.float32),
            jax.ShapeDtypeStruct((2, c2, 128), jnp.float32),
        ],
    )(h1, st1, g1, bb1, conv2_w, b2)

    out = pl.pallas_call(
        functools.partial(_bn2_body, count=count),
        grid=grid,
        in_specs=[
            pl.BlockSpec((1, c2, tn), lambda i, j: (i, 0, j)),
            pl.BlockSpec((2, c2, 128), lambda i, j: (0, 0, 0)),
            pl.BlockSpec((c2, 1), lambda i, j: (0, 0)),
            pl.BlockSpec((c2, 1), lambda i, j: (0, 0)),
        ],
        out_specs=pl.BlockSpec((1, c2, tn), lambda i, j: (i, 0, j)),
        out_shape=jax.ShapeDtypeStruct((b, c2, n), jnp.float32),
    )(h2, st2, g2, bb2)

    return out
